# baseline (device time: 48864 ns/iter reference)
import jax
import jax.numpy as jnp
from jax import lax
from jax.experimental import pallas as pl
from jax.experimental.pallas import tpu as pltpu

N_DEV = 4
WIRE_DTYPE = jnp.bfloat16
GROUPS = [(0, 512), (512, 512)]
G = len(GROUPS)


def kernel(x):
    m_per, n = x.shape
    mh = m_per // 2
    ms = mh // 2
    mq = mh // 4

    def body(x_ref, out_ref, xb, r1, r2, p, s2, s3, send_sems, recv_sems):
        my = lax.axis_index("i")
        py = my + 1 - 2 * lax.rem(my, 2)
        px = 3 - my
        bit_y = lax.rem((my + 1) // 2, 2)
        bit_x = my // 2

        barrier_sem = pltpu.get_barrier_semaphore()
        for nbr in (py, px):
            pl.semaphore_signal(
                barrier_sem, inc=1,
                device_id=(nbr,), device_id_type=pl.DeviceIdType.MESH,
            )

        def rdma(q, g, src, dst, tgt):
            i = q * G + g
            return pltpu.make_async_remote_copy(
                src_ref=src, dst_ref=dst,
                send_sem=send_sems.at[i], recv_sem=recv_sems.at[i],
                device_id=(tgt,), device_id_type=pl.DeviceIdType.MESH,
            )

        last = {"y": None, "x": None}

        def chain_start(link, desc):
            if last[link] is not None:
                last[link].wait_send()
            desc.start()
            last[link] = desc

        offa1 = (1 - bit_y) * ms
        offb1 = mh + (1 - bit_x) * ms
        keep_a = bit_y * ms
        keep_b = mh + bit_x * ms
        qa1 = (1 - bit_x) * mq
        qb1 = (1 - bit_y) * mq
        qa2 = bit_x * mq
        qb2 = bit_y * mq
        offa = bit_y * ms + bit_x * mq
        offb = mh + bit_x * ms + bit_y * mq
        offa_p = bit_y * ms + (1 - bit_x) * mq
        offb_p = mh + bit_x * ms + (1 - bit_y) * mq

        ex = {}
        csg = [pl.ds(o, s) for o, s in GROUPS]

        xb[pl.ds(offa1, ms), csg[0]] = (
            x_ref[pl.ds(offa1, ms), csg[0]].astype(WIRE_DTYPE)
        )
        xb[pl.ds(offb1, ms), csg[0]] = (
            x_ref[pl.ds(offb1, ms), csg[0]].astype(WIRE_DTYPE)
        )
        pl.semaphore_wait(barrier_sem, 2)

        cs = csg[0]
        ex[0, 0] = rdma(0, 0, xb.at[pl.ds(offa1, ms), cs], r1.at[0, :, cs], py)
        ex[1, 0] = rdma(1, 0, xb.at[pl.ds(offb1, ms), cs], r1.at[1, :, cs], px)
        chain_start("y", ex[0, 0])
        chain_start("x", ex[1, 0])
        cs = csg[1]
        xb[pl.ds(offa1, ms), cs] = x_ref[pl.ds(offa1, ms), cs].astype(WIRE_DTYPE)
        xb[pl.ds(offb1, ms), cs] = x_ref[pl.ds(offb1, ms), cs].astype(WIRE_DTYPE)
        for g in range(G):
            xb[pl.ds(keep_a, ms), csg[g]] = (
                x_ref[pl.ds(keep_a, ms), csg[g]].astype(WIRE_DTYPE)
            )
            xb[pl.ds(keep_b, ms), csg[g]] = (
                x_ref[pl.ds(keep_b, ms), csg[g]].astype(WIRE_DTYPE)
            )
        ex[0, 1] = rdma(0, 1, xb.at[pl.ds(offa1, ms), cs], r1.at[0, :, cs], py)
        ex[1, 1] = rdma(1, 1, xb.at[pl.ds(offb1, ms), cs], r1.at[1, :, cs], px)
        chain_start("y", ex[0, 1])
        chain_start("x", ex[1, 1])

        for g in range(G):
            cs = csg[g]
            ex[0, g].wait_recv()
            s2[0, :, cs] = (
                r1[0, pl.ds(qa1, mq), cs] + xb[pl.ds(keep_a + qa1, mq), cs]
            )
            ex[1, g].wait_recv()
            s2[1, :, cs] = (
                r1[1, pl.ds(qb1, mq), cs] + xb[pl.ds(keep_b + qb1, mq), cs]
            )
            ex[2, g] = rdma(2, g, s2.at[0, :, cs], r2.at[0, :, cs], px)
            ex[3, g] = rdma(3, g, s2.at[1, :, cs], r2.at[1, :, cs], py)
            chain_start("x", ex[2, g])
            chain_start("y", ex[3, g])
            p[0, :, cs] = (
                r1[0, pl.ds(qa2, mq), cs] + xb[pl.ds(keep_a + qa2, mq), cs]
            )
            p[1, :, cs] = (
                r1[1, pl.ds(qb2, mq), cs] + xb[pl.ds(keep_b + qb2, mq), cs]
            )

        for g in range(G):
            cs = csg[g]
            ex[2, g].wait_recv()
            red_a = r2[0, :, cs] + p[0, :, cs]
            s3[0, :, cs] = red_a
            ex[3, g].wait_recv()
            red_b = r2[1, :, cs] + p[1, :, cs]
            s3[1, :, cs] = red_b
            ex[4, g] = rdma(4, g, s3.at[0, :, cs],
                            out_ref.at[pl.ds(offa, mq), cs], px)
            ex[5, g] = rdma(5, g, s3.at[1, :, cs],
                            out_ref.at[pl.ds(offb, mq), cs], py)
            ex[6, g] = rdma(6, g, s3.at[0, :, cs],
                            out_ref.at[pl.ds(offa, mq), cs], py)
            ex[8, g] = rdma(8, g, s3.at[1, :, cs],
                            out_ref.at[pl.ds(offb, mq), cs], px)
            chain_start("x", ex[4, g])
            chain_start("y", ex[5, g])
            chain_start("y", ex[6, g])
            chain_start("x", ex[8, g])
            out_ref[pl.ds(offa, mq), cs] = red_a
            out_ref[pl.ds(offb, mq), cs] = red_b

        for g in range(G):
            cs = csg[g]
            ex[4, g].wait_recv()
            ex[7, g] = rdma(7, g, out_ref.at[pl.ds(offa_p, mq), cs],
                            out_ref.at[pl.ds(offa_p, mq), cs], py)
            chain_start("y", ex[7, g])
            ex[5, g].wait_recv()
            ex[9, g] = rdma(9, g, out_ref.at[pl.ds(offb_p, mq), cs],
                            out_ref.at[pl.ds(offb_p, mq), cs], px)
            chain_start("x", ex[9, g])

        for g in range(G):
            for q in (6, 7, 8, 9):
                ex[q, g].wait_recv()

        last["y"].wait_send()
        last["x"].wait_send()

    return pl.pallas_call(
        body,
        out_shape=jax.ShapeDtypeStruct((m_per, n), WIRE_DTYPE),
        in_specs=[pl.BlockSpec(memory_space=pltpu.VMEM)],
        out_specs=pl.BlockSpec(memory_space=pltpu.VMEM),
        scratch_shapes=[
            pltpu.VMEM((m_per, n), WIRE_DTYPE),
            pltpu.VMEM((2, ms, n), WIRE_DTYPE),
            pltpu.VMEM((2, mq, n), WIRE_DTYPE),
            pltpu.VMEM((2, mq, n), WIRE_DTYPE),
            pltpu.VMEM((2, mq, n), WIRE_DTYPE),
            pltpu.VMEM((2, mq, n), WIRE_DTYPE),
            pltpu.SemaphoreType.DMA((10 * G,)),
            pltpu.SemaphoreType.DMA((10 * G,)),
        ],
        compiler_params=pltpu.CompilerParams(collective_id=0),
    )(x)


# device time: 43420 ns/iter; 1.1254x vs baseline; 1.1254x over previous
import jax
import jax.numpy as jnp
from jax import lax
from jax.experimental import pallas as pl
from jax.experimental.pallas import tpu as pltpu

N_DEV = 4
WIRE_DTYPE = jnp.bfloat16
CHUNKS = [(0, 128), (128, 384), (512, 512)]
C = len(CHUNKS)


def kernel(x):
    m_per, n = x.shape
    mh = m_per // 2
    ms = mh // 2
    mq = mh // 4

    def body(x_ref, out_ref, xb, r1, r2, p, s2, s3, send_sems, recv_sems):
        my = lax.axis_index("i")
        py = my + 1 - 2 * lax.rem(my, 2)
        px = 3 - my
        bit_y = lax.rem((my + 1) // 2, 2)
        bit_x = my // 2

        barrier_sem = pltpu.get_barrier_semaphore()
        for nbr in (py, px):
            pl.semaphore_signal(
                barrier_sem, inc=1,
                device_id=(nbr,), device_id_type=pl.DeviceIdType.MESH,
            )

        def rdma(q, c, src, dst, tgt):
            i = q * C + c
            return pltpu.make_async_remote_copy(
                src_ref=src, dst_ref=dst,
                send_sem=send_sems.at[i], recv_sem=recv_sems.at[i],
                device_id=(tgt,), device_id_type=pl.DeviceIdType.MESH,
            )

        offa1 = (1 - bit_y) * ms
        offb1 = mh + (1 - bit_x) * ms
        keep_a = bit_y * ms
        keep_b = mh + bit_x * ms
        qa1 = (1 - bit_x) * mq
        qb1 = (1 - bit_y) * mq
        qa2 = bit_x * mq
        qb2 = bit_y * mq
        offa = bit_y * ms + bit_x * mq
        offb = mh + bit_x * ms + bit_y * mq
        offa_p = bit_y * ms + (1 - bit_x) * mq
        offb_p = mh + bit_x * ms + (1 - bit_y) * mq

        ex = {}

        cs0 = pl.ds(CHUNKS[0][0], CHUNKS[0][1])
        xb[pl.ds(offa1, ms), cs0] = x_ref[pl.ds(offa1, ms), cs0].astype(WIRE_DTYPE)
        xb[pl.ds(offb1, ms), cs0] = x_ref[pl.ds(offb1, ms), cs0].astype(WIRE_DTYPE)
        pl.semaphore_wait(barrier_sem, 2)

        for c, (co, cn) in enumerate(CHUNKS):
            cs = pl.ds(co, cn)
            if c > 0:
                xb[pl.ds(offa1, ms), cs] = (
                    x_ref[pl.ds(offa1, ms), cs].astype(WIRE_DTYPE)
                )
                xb[pl.ds(offb1, ms), cs] = (
                    x_ref[pl.ds(offb1, ms), cs].astype(WIRE_DTYPE)
                )
            ex[0, c] = rdma(0, c, xb.at[pl.ds(offa1, ms), cs],
                            r1.at[0, :, cs], py)
            ex[1, c] = rdma(1, c, xb.at[pl.ds(offb1, ms), cs],
                            r1.at[1, :, cs], px)
            ex[0, c].start()
            ex[1, c].start()
        for c, (co, cn) in enumerate(CHUNKS):
            cs = pl.ds(co, cn)
            xb[pl.ds(keep_a, ms), cs] = (
                x_ref[pl.ds(keep_a, ms), cs].astype(WIRE_DTYPE)
            )
            xb[pl.ds(keep_b, ms), cs] = (
                x_ref[pl.ds(keep_b, ms), cs].astype(WIRE_DTYPE)
            )

        for c, (co, cn) in enumerate(CHUNKS):
            cs = pl.ds(co, cn)
            ex[0, c].wait_recv()
            s2[0, :, cs] = (
                r1[0, pl.ds(qa1, mq), cs] + xb[pl.ds(keep_a + qa1, mq), cs]
            )
            ex[2, c] = rdma(2, c, s2.at[0, :, cs], r2.at[0, :, cs], px)
            ex[2, c].start()

            ex[1, c].wait_recv()
            s2[1, :, cs] = (
                r1[1, pl.ds(qb1, mq), cs] + xb[pl.ds(keep_b + qb1, mq), cs]
            )
            ex[3, c] = rdma(3, c, s2.at[1, :, cs], r2.at[1, :, cs], py)
            ex[3, c].start()

            p[0, :, cs] = (
                r1[0, pl.ds(qa2, mq), cs] + xb[pl.ds(keep_a + qa2, mq), cs]
            )
            p[1, :, cs] = (
                r1[1, pl.ds(qb2, mq), cs] + xb[pl.ds(keep_b + qb2, mq), cs]
            )

        for c, (co, cn) in enumerate(CHUNKS):
            cs = pl.ds(co, cn)
            ex[2, c].wait_recv()
            red_a = r2[0, :, cs] + p[0, :, cs]
            s3[0, :, cs] = red_a
            ex[4, c] = rdma(4, c, s3.at[0, :, cs],
                            out_ref.at[pl.ds(offa, mq), cs], px)
            ex[6, c] = rdma(6, c, s3.at[0, :, cs],
                            out_ref.at[pl.ds(offa, mq), cs], py)
            ex[4, c].start()
            ex[6, c].start()
            out_ref[pl.ds(offa, mq), cs] = red_a

            ex[3, c].wait_recv()
            red_b = r2[1, :, cs] + p[1, :, cs]
            s3[1, :, cs] = red_b
            ex[5, c] = rdma(5, c, s3.at[1, :, cs],
                            out_ref.at[pl.ds(offb, mq), cs], py)
            ex[8, c] = rdma(8, c, s3.at[1, :, cs],
                            out_ref.at[pl.ds(offb, mq), cs], px)
            ex[5, c].start()
            ex[8, c].start()
            out_ref[pl.ds(offb, mq), cs] = red_b

        for c, (co, cn) in enumerate(CHUNKS):
            cs = pl.ds(co, cn)
            ex[4, c].wait_recv()
            ex[7, c] = rdma(7, c, out_ref.at[pl.ds(offa_p, mq), cs],
                            out_ref.at[pl.ds(offa_p, mq), cs], py)
            ex[7, c].start()

            ex[5, c].wait_recv()
            ex[9, c] = rdma(9, c, out_ref.at[pl.ds(offb_p, mq), cs],
                            out_ref.at[pl.ds(offb_p, mq), cs], px)
            ex[9, c].start()

        for c in range(C):
            for q in (6, 7, 8, 9):
                ex[q, c].wait_recv()

        for key in ex:
            ex[key].wait_send()

    return pl.pallas_call(
        body,
        out_shape=jax.ShapeDtypeStruct((m_per, n), WIRE_DTYPE),
        in_specs=[pl.BlockSpec(memory_space=pltpu.VMEM)],
        out_specs=pl.BlockSpec(memory_space=pltpu.VMEM),
        scratch_shapes=[
            pltpu.VMEM((m_per, n), WIRE_DTYPE),
            pltpu.VMEM((2, ms, n), WIRE_DTYPE),
            pltpu.VMEM((2, mq, n), WIRE_DTYPE),
            pltpu.VMEM((2, mq, n), WIRE_DTYPE),
            pltpu.VMEM((2, mq, n), WIRE_DTYPE),
            pltpu.VMEM((2, mq, n), WIRE_DTYPE),
            pltpu.SemaphoreType.DMA((10 * C,)),
            pltpu.SemaphoreType.DMA((10 * C,)),
        ],
        compiler_params=pltpu.CompilerParams(collective_id=0),
    )(x)
